# baseline (speedup 1.0000x reference)
import jax
import jax.numpy as jnp
from jax import lax
from jax.experimental import pallas as pl
from jax.experimental.pallas import tpu as pltpu
from jax.experimental.pallas import tpu_sc as plsc

_NUM_BINS = 100
_DIM = 128
_LANES = 16


def _feature_lookup(num_hbm, table_hbm, out_hbm, num_v, rows_v, sem):
    cid = lax.axis_index("c")
    sid = lax.axis_index("s")

    @pl.when((cid == 0) & (sid == 0))
    def _():
        pltpu.sync_copy(num_hbm, num_v)
        nv = num_v[...]
        in_range = (nv >= 0) & (nv < _NUM_BINS)
        idx = jnp.where(in_range, nv, jnp.zeros_like(nv))
        pltpu.async_copy(table_hbm.at[idx], rows_v, sem).wait()
        pltpu.sync_copy(rows_v.at[0], out_hbm)


def kernel(num, table):
    num16 = jnp.full((_LANES,), num, dtype=jnp.int32)
    call = pl.kernel(
        _feature_lookup,
        mesh=plsc.VectorSubcoreMesh(core_axis_name="c", subcore_axis_name="s"),
        out_type=jax.ShapeDtypeStruct((_DIM,), jnp.float32),
        scratch_types=[
            pltpu.VMEM((_LANES,), jnp.int32),
            pltpu.VMEM((_LANES, _DIM), jnp.float32),
            pltpu.SemaphoreType.DMA,
        ],
    )
    return call(num16, table)


# speedup vs baseline: 1.0943x; 1.0943x over previous
import jax
import jax.numpy as jnp
from jax import lax
from jax.experimental import pallas as pl
from jax.experimental.pallas import tpu as pltpu
from jax.experimental.pallas import tpu_sc as plsc

_NUM_BINS = 100
_DIM = 128
_LANES = 16


def _feature_lookup(num_hbm, table_hbm, out_hbm, num_v, rows_v, sem):
    cid = lax.axis_index("c")
    sid = lax.axis_index("s")

    @pl.when((cid == 0) & (sid == 0))
    def _():
        pltpu.sync_copy(num_hbm, num_v)
        nv = num_v[...]
        in_range = (nv >= 0) & (nv < _NUM_BINS)
        idx = jnp.where(in_range, nv, jnp.zeros_like(nv))
        pltpu.async_copy(table_hbm.at[idx], rows_v, sem).wait()
        pltpu.sync_copy(rows_v.at[0], out_hbm)


def kernel(num, table):
    num16 = jnp.full((_LANES,), num, dtype=jnp.int32)
    call = pl.kernel(
        _feature_lookup,
        mesh=plsc.VectorSubcoreMesh(
            core_axis_name="c", subcore_axis_name="s", num_cores=1
        ),
        out_type=jax.ShapeDtypeStruct((_DIM,), jnp.float32),
        scratch_types=[
            pltpu.VMEM((_LANES,), jnp.int32),
            pltpu.VMEM((_LANES, _DIM), jnp.float32),
            pltpu.SemaphoreType.DMA,
        ],
    )
    return call(num16, table)


# speedup vs baseline: 1.2721x; 1.1625x over previous
import jax
import jax.numpy as jnp
from jax import lax
from jax.experimental import pallas as pl
from jax.experimental.pallas import tpu as pltpu
from jax.experimental.pallas import tpu_sc as plsc

_NUM_BINS = 100
_DIM = 128


def _feature_lookup(num_hbm, table_hbm, out_hbm, num_s):
    cid = lax.axis_index("c")

    @pl.when(cid == 0)
    def _():
        pltpu.sync_copy(num_hbm, num_s)
        n = num_s[0]
        idx = jnp.where((n >= 0) & (n < _NUM_BINS), n, 0)
        pltpu.sync_copy(table_hbm.at[idx], out_hbm)


def kernel(num, table):
    num1 = jnp.asarray(num, dtype=jnp.int32).reshape(1)
    call = pl.kernel(
        _feature_lookup,
        mesh=plsc.ScalarSubcoreMesh(axis_name="c", num_cores=1),
        out_type=jax.ShapeDtypeStruct((_DIM,), jnp.float32),
        scratch_types=[pltpu.SMEM((1,), jnp.int32)],
    )
    return call(num1, table)


# speedup vs baseline: 1.2757x; 1.0029x over previous
import jax
import jax.numpy as jnp
from jax import lax
from jax.experimental import pallas as pl
from jax.experimental.pallas import tpu as pltpu
from jax.experimental.pallas import tpu_sc as plsc

_NUM_BINS = 100
_DIM = 128


def _feature_lookup(num_hbm, table_hbm, out_hbm, num_s):
    pltpu.sync_copy(num_hbm, num_s)
    n = num_s[0]
    idx = jnp.where((n >= 0) & (n < _NUM_BINS), n, 0)
    pltpu.sync_copy(table_hbm.at[idx], out_hbm)


def kernel(num, table):
    num1 = jnp.asarray(num, dtype=jnp.int32).reshape(1)
    call = pl.kernel(
        _feature_lookup,
        mesh=plsc.ScalarSubcoreMesh(axis_name="c", num_cores=1),
        out_type=jax.ShapeDtypeStruct((_DIM,), jnp.float32),
        scratch_types=[pltpu.SMEM((1,), jnp.int32)],
    )
    return call(num1, table)


# speedup vs baseline: 1.2775x; 1.0014x over previous
import jax
import jax.numpy as jnp
from jax.experimental import pallas as pl
from jax.experimental.pallas import tpu as pltpu
from jax.experimental.pallas import tpu_sc as plsc

_NUM_BINS = 100
_DIM = 128


def _feature_lookup(num_hbm, table_hbm, out_hbm, num_s):
    pltpu.sync_copy(num_hbm, num_s)
    n = num_s[0]
    idx = jnp.where((n >= 0) & (n < _NUM_BINS), n, 0)
    pltpu.sync_copy(table_hbm.at[idx], out_hbm)


def kernel(num, table):
    num1 = jnp.asarray(num, dtype=jnp.int32).reshape(1)
    call = pl.kernel(
        _feature_lookup,
        mesh=plsc.ScalarSubcoreMesh(axis_name="c", num_cores=1),
        out_type=jax.ShapeDtypeStruct((_DIM,), jnp.float32),
        scratch_types=[pltpu.SMEM((1,), jnp.int32)],
    )
    return call(num1, table)
